# baseline (speedup 1.0000x reference)
import functools

import jax
import jax.numpy as jnp
from jax import lax
from jax.experimental import pallas as pl
from jax.experimental.pallas import tpu as pltpu
from jax.experimental.pallas import tpu_sc as plsc

_D = 32
_K = 8192
_N = 32768
_MB = 512
_EPS = 1e-5

_NC = 2
_NS = 16
_NW = _NC * _NS
_BPW = _N // _NW


def _argmin_body(x_ref, emb_ref, cs_ref, ind_ref, ppl_ref, ee_ref):
    i = pl.program_id(0)

    @pl.when(i == 0)
    def _init():
        e = emb_ref[...]
        ee_ref[...] = lax.dot_general(
            jnp.ones((1, _D), jnp.float32), e * e,
            (((1,), (1,)), ((), ())), preferred_element_type=jnp.float32,
            precision=lax.Precision.HIGHEST)
        p = cs_ref[...]
        ppl_ref[0, 0] = jnp.exp(-jnp.sum(p * jnp.log(p + _EPS)))

    xb = x_ref[...]
    xx = jnp.sum(xb * xb, axis=1, keepdims=True)
    lhs = jnp.concatenate([xb + xb, xx], axis=1).astype(jnp.bfloat16)
    rhs = jnp.concatenate([-emb_ref[...], jnp.ones((_K, 1), jnp.float32)],
                          axis=1).astype(jnp.bfloat16)
    s_halves = [
        lax.dot_general(lhs, rhs[hb:hb + _K // 2, :],
                        (((1,), (1,)), ((), ())),
                        preferred_element_type=jnp.float32)
        for hb in (0, _K // 2)
    ]
    dxx = (xx - xx.astype(jnp.bfloat16).astype(jnp.float32))[:, 0]
    ee = ee_ref[...]
    h = _K // 2
    nch = h // 128
    _RB = 128
    lane = lax.broadcasted_iota(jnp.int32, (_RB, 128), 1)

    def scan_half(rbase, hbase):
        sh = s_halves[hbase // (_K // 2)]

        def chunk_t(base):
            return (sh[rbase:rbase + _RB, base - hbase:base - hbase + 128]
                    + ee[:, base:base + 128])
        acc_v = chunk_t(hbase)
        acc_c = jnp.zeros_like(lane)
        for c in range(1, nch):
            v = chunk_t(hbase + c * 128)
            lt = v < acc_v
            acc_c = jnp.where(lt, c, acc_c)
            acc_v = jnp.where(lt, v, acc_v)
        m = jnp.min(acc_v, axis=1, keepdims=True)
        idx = acc_c * 128 + lane
        i = jnp.min(jnp.where(acc_v == m, idx, _K), axis=1)
        return m[:, 0], i

    for rb in range(0, _MB, _RB):
        tm0, i0 = scan_half(rb, 0)
        tm1, i1 = scan_half(rb, h)
        d = dxx[rb:rb + _RB]
        tm0b = (tm0 + d).astype(jnp.bfloat16).astype(jnp.float32)
        ind_ref[0, 0, rb:rb + _RB] = jnp.where(tm1 + d < tm0b, i1 + h, i0)


_argmin_call = pl.pallas_call(
    _argmin_body,
    grid=(_N // _MB,),
    in_specs=[
        pl.BlockSpec((_MB, _D), lambda i: (i, 0)),
        pl.BlockSpec((_K, _D), lambda i: (0, 0)),
        pl.BlockSpec((8, _K // 8), lambda i: (0, 0)),
    ],
    out_specs=[
        pl.BlockSpec((1, 1, _MB), lambda i: (i, 0, 0)),
        pl.BlockSpec(memory_space=pltpu.SMEM),
    ],
    out_shape=[
        jax.ShapeDtypeStruct((_N // _MB, 1, _MB), jnp.int32),
        jax.ShapeDtypeStruct((1, 1), jnp.float32),
    ],
    scratch_shapes=[pltpu.VMEM((1, _K), jnp.float32)],
    compiler_params=pltpu.CompilerParams(
        dimension_semantics=("arbitrary",)),
)


@functools.cache
def _gather_rows_call():
    mesh = plsc.VectorSubcoreMesh(
        core_axis_name="c", subcore_axis_name="s",
        num_cores=_NC, num_subcores=_NS)

    @functools.partial(
        pl.kernel,
        mesh=mesh,
        out_type=jax.ShapeDtypeStruct((_N, _D), jnp.float32),
        scratch_types=[
            pltpu.VMEM((_BPW,), jnp.int32),
            pltpu.VMEM((_BPW, _D), jnp.float32),
            pltpu.SemaphoreType.DMA,
        ],
        compiler_params=pltpu.CompilerParams(use_tc_tiling_on_sc=False),
    )
    def _gather_rows(idx_hbm, table_hbm, out_hbm, idx_v, rows_v, sem):
        wid = lax.axis_index("s") * _NC + lax.axis_index("c")
        base = wid * _BPW
        pltpu.sync_copy(idx_hbm.at[pl.ds(base, _BPW)], idx_v)
        pltpu.async_copy(table_hbm.at[idx_v], rows_v, sem).wait()
        pltpu.sync_copy(rows_v, out_hbm.at[pl.ds(base, _BPW)])

    return _gather_rows


def kernel(x, embed, cluster_size):
    shape = x.shape
    flat = x.astype(jnp.float32).reshape(-1, shape[-1])
    ind3, ppl = _argmin_call(flat, embed, cluster_size.reshape(8, _K // 8))
    ind_flat = ind3.reshape(-1)
    quantize = _gather_rows_call()(ind_flat, embed)
    return (quantize.reshape(shape), ind_flat.reshape(shape[:-1]), ppl[0, 0])


# speedup vs baseline: 1.0721x; 1.0721x over previous
import functools

import jax
import jax.numpy as jnp
from jax import lax
from jax.experimental import pallas as pl
from jax.experimental.pallas import tpu as pltpu
from jax.experimental.pallas import tpu_sc as plsc

_D = 32
_K = 8192
_N = 32768
_MB = 1024
_EPS = 1e-5

_NC = 2
_NS = 16
_NW = _NC * _NS
_BPW = _N // _NW


def _argmin_body(x_ref, emb_ref, cs_ref, ind_ref, ppl_ref, ee_ref):
    i = pl.program_id(0)

    @pl.when(i == 0)
    def _init():
        e = emb_ref[...]
        ee_ref[...] = lax.dot_general(
            jnp.ones((1, _D), jnp.float32), e * e,
            (((1,), (1,)), ((), ())), preferred_element_type=jnp.float32,
            precision=lax.Precision.HIGHEST)
        p = cs_ref[...]
        ppl_ref[0, 0] = jnp.exp(-jnp.sum(p * jnp.log(p + _EPS)))

    xb = x_ref[...]
    xx = jnp.sum(xb * xb, axis=1, keepdims=True)
    lhs = jnp.concatenate([xb + xb, xx], axis=1).astype(jnp.bfloat16)
    rhs = jnp.concatenate([-emb_ref[...], jnp.ones((_K, 1), jnp.float32)],
                          axis=1).astype(jnp.bfloat16)
    s_halves = [
        lax.dot_general(lhs, rhs[hb:hb + _K // 2, :],
                        (((1,), (1,)), ((), ())),
                        preferred_element_type=jnp.float32)
        for hb in (0, _K // 2)
    ]
    dxx = (xx - xx.astype(jnp.bfloat16).astype(jnp.float32))[:, 0]
    ee = ee_ref[...]
    h = _K // 2
    nch = h // 128
    _RB = 128
    lane = lax.broadcasted_iota(jnp.int32, (_RB, 128), 1)

    def scan_half(rbase, hbase):
        sh = s_halves[hbase // (_K // 2)]

        def chunk_t(base):
            return (sh[rbase:rbase + _RB, base - hbase:base - hbase + 128]
                    + ee[:, base:base + 128])
        acc_v = chunk_t(hbase)
        acc_c = jnp.zeros_like(lane)
        for c in range(1, nch):
            v = chunk_t(hbase + c * 128)
            lt = v < acc_v
            acc_c = jnp.where(lt, c, acc_c)
            acc_v = jnp.where(lt, v, acc_v)
        m = jnp.min(acc_v, axis=1, keepdims=True)
        idx = acc_c * 128 + lane
        i = jnp.min(jnp.where(acc_v == m, idx, _K), axis=1)
        return m[:, 0], i

    for rb in range(0, _MB, _RB):
        tm0, i0 = scan_half(rb, 0)
        tm1, i1 = scan_half(rb, h)
        d = dxx[rb:rb + _RB]
        tm0b = (tm0 + d).astype(jnp.bfloat16).astype(jnp.float32)
        ind_ref[0, 0, rb:rb + _RB] = jnp.where(tm1 + d < tm0b, i1 + h, i0)


_argmin_call = pl.pallas_call(
    _argmin_body,
    grid=(_N // _MB,),
    in_specs=[
        pl.BlockSpec((_MB, _D), lambda i: (i, 0)),
        pl.BlockSpec((_K, _D), lambda i: (0, 0)),
        pl.BlockSpec((8, _K // 8), lambda i: (0, 0)),
    ],
    out_specs=[
        pl.BlockSpec((1, 1, _MB), lambda i: (i, 0, 0)),
        pl.BlockSpec(memory_space=pltpu.SMEM),
    ],
    out_shape=[
        jax.ShapeDtypeStruct((_N // _MB, 1, _MB), jnp.int32),
        jax.ShapeDtypeStruct((1, 1), jnp.float32),
    ],
    scratch_shapes=[pltpu.VMEM((1, _K), jnp.float32)],
    compiler_params=pltpu.CompilerParams(
        dimension_semantics=("arbitrary",)),
)


@functools.cache
def _gather_rows_call():
    mesh = plsc.VectorSubcoreMesh(
        core_axis_name="c", subcore_axis_name="s",
        num_cores=_NC, num_subcores=_NS)

    @functools.partial(
        pl.kernel,
        mesh=mesh,
        out_type=jax.ShapeDtypeStruct((_N, _D), jnp.float32),
        scratch_types=[
            pltpu.VMEM((_BPW,), jnp.int32),
            pltpu.VMEM((_BPW, _D), jnp.float32),
            pltpu.SemaphoreType.DMA,
        ],
        compiler_params=pltpu.CompilerParams(use_tc_tiling_on_sc=False),
    )
    def _gather_rows(idx_hbm, table_hbm, out_hbm, idx_v, rows_v, sem):
        wid = lax.axis_index("s") * _NC + lax.axis_index("c")
        base = wid * _BPW
        pltpu.sync_copy(idx_hbm.at[pl.ds(base, _BPW)], idx_v)
        pltpu.async_copy(table_hbm.at[idx_v], rows_v, sem).wait()
        pltpu.sync_copy(rows_v, out_hbm.at[pl.ds(base, _BPW)])

    return _gather_rows


def kernel(x, embed, cluster_size):
    shape = x.shape
    flat = x.astype(jnp.float32).reshape(-1, shape[-1])
    ind3, ppl = _argmin_call(flat, embed, cluster_size.reshape(8, _K // 8))
    ind_flat = ind3.reshape(-1)
    quantize = _gather_rows_call()(ind_flat, embed)
    return (quantize.reshape(shape), ind_flat.reshape(shape[:-1]), ppl[0, 0])
